# Initial kernel scaffold; baseline (speedup 1.0000x reference)
import jax
import jax.numpy as jnp
from jax.experimental import pallas as pl


def kernel(x, edge_index, Wp, bp, gp, bep, Wa, ba, ga, bea, Wo, bo, go, beo):
    raise NotImplementedError("write your pallas kernel here")


# speedup vs baseline: 3.1680x; 3.1680x over previous
import functools

import jax
import jax.numpy as jnp
from jax import lax
from jax.experimental import pallas as pl
from jax.experimental.pallas import tpu as pltpu
from jax.experimental.pallas import tpu_sc as plsc

MOT = 256
HALF = 128
NC = 2
NS = 16
CHUNK = 80
IW = 16
NCHUNK = 256
ROWBLK = 1000


def _ln(y, g, b, eps=1e-5):
    m = jnp.mean(y, axis=-1, keepdims=True)
    v = jnp.mean((y - m) ** 2, axis=-1, keepdims=True)
    return (y - m) * jax.lax.rsqrt(v + eps) * g + b


def _gelu(y):
    return 0.5 * y * (1.0 + lax.erf(y * (2.0 ** -0.5)))


def _make_agg(n_pad):
    rows_per_tile = n_pad // NS
    nw = NCHUNK // IW

    mesh = plsc.VectorSubcoreMesh(core_axis_name="c", subcore_axis_name="s")

    def body(h0, h1, colr, rowr, zeros, a0, a1,
             colw, roww, gbuf, aggsp, isem, gsem, ssem):
        c = lax.axis_index("c")
        s = lax.axis_index("s")
        sl = pl.ds(s * rows_per_tile, rows_per_tile)
        pltpu.sync_copy(zeros, aggsp.at[sl])
        plsc.subcore_barrier()

        def idx_dma(w, wb):
            return (
                pltpu.make_async_copy(colr.at[s, pl.ds(w * IW, IW)], colw.at[wb], isem),
                pltpu.make_async_copy(rowr.at[s, pl.ds(w * IW, IW)], roww.at[wb], isem),
            )

        def run(h):
            def gather(wb, i):
                return pltpu.make_async_copy(
                    h.at[colw.at[wb, i]], gbuf.at[i % 2], gsem)

            def scatter(wb, i):
                return pltpu.make_async_copy(
                    gbuf.at[i % 2], aggsp.at[roww.at[wb, i]], ssem)

            for d in idx_dma(0, 0):
                d.start()

            def window(w, carry):
                wb = lax.rem(w, 2)
                for d in idx_dma(w, wb):
                    d.wait()

                @pl.when(w + 1 < nw)
                def _():
                    for d in idx_dma(w + 1, 1 - wb):
                        d.start()

                gather(wb, 0).start()
                gather(wb, 1).start()
                for i in range(IW):
                    gather(wb, i).wait()
                    sc = scatter(wb, i)
                    sc.start(add=True)
                    sc.wait()
                    if i + 2 < IW:
                        gather(wb, i + 2).start()
                return carry

            lax.fori_loop(0, nw, window, 0, unroll=False)

        @pl.when(c == 0)
        def _():
            run(h0)

        @pl.when(c == 1)
        def _():
            run(h1)

        plsc.subcore_barrier()

        @pl.when(c == 0)
        def _():
            pltpu.sync_copy(aggsp.at[sl], a0.at[sl])

        @pl.when(c == 1)
        def _():
            pltpu.sync_copy(aggsp.at[sl], a1.at[sl])

    out = jax.ShapeDtypeStruct((n_pad, HALF), jnp.float32)
    return pl.kernel(
        body,
        out_type=[out, out],
        mesh=mesh,
        scratch_types=[
            pltpu.VMEM((2, IW, CHUNK), jnp.int32),
            pltpu.VMEM((2, IW, CHUNK), jnp.int32),
            pltpu.VMEM((2, CHUNK, HALF), jnp.float32),
            pltpu.VMEM_SHARED((n_pad, HALF), jnp.float32),
            pltpu.SemaphoreType.DMA,
            pltpu.SemaphoreType.DMA,
            pltpu.SemaphoreType.DMA,
        ],
    )


def _proj_body(xb, Wp, bp, gp, bep, h0, h1):
    y = jnp.dot(xb[...], Wp[...], preferred_element_type=jnp.float32)
    y = _gelu(_ln(y + bp[...], gp[...], bep[...]))
    h0[...] = y[:, :HALF]
    h1[...] = y[:, HALF:]


def _proj(x, Wp, bp, gp, bep):
    n, hid = x.shape
    grid = n // ROWBLK
    out = jax.ShapeDtypeStruct((n, HALF), jnp.float32)
    hblk = pl.BlockSpec((ROWBLK, HALF), lambda i: (i, 0))
    vblk = pl.BlockSpec((1, MOT), lambda i: (0, 0))
    return pl.pallas_call(
        _proj_body,
        grid=(grid,),
        in_specs=[
            pl.BlockSpec((ROWBLK, hid), lambda i: (i, 0)),
            pl.BlockSpec((hid, MOT), lambda i: (0, 0)),
            vblk, vblk, vblk,
        ],
        out_specs=[hblk, hblk],
        out_shape=[out, out],
    )(x, Wp, bp, gp, bep)


def _step_body(h0b, h1b, a0b, a1b, W1, W2, W3, W4, b, g, be, o0, o1):
    y = jnp.dot(h0b[...], W1[...], preferred_element_type=jnp.float32)
    y += jnp.dot(h1b[...], W2[...], preferred_element_type=jnp.float32)
    y += jnp.dot(a0b[...], W3[...], preferred_element_type=jnp.float32)
    y += jnp.dot(a1b[...], W4[...], preferred_element_type=jnp.float32)
    y = _gelu(_ln(y + b[...], g[...], be[...]))
    o0[...] = y[:, :HALF]
    o1[...] = y[:, HALF:]


def _step(h0, h1, a0, a1, W1, W2, W3, W4, b, g, be):
    n = h0.shape[0]
    grid = n // ROWBLK
    hblk = pl.BlockSpec((ROWBLK, HALF), lambda i: (i, 0))
    wblk = pl.BlockSpec((HALF, MOT), lambda i: (0, 0))
    vblk = pl.BlockSpec((1, MOT), lambda i: (0, 0))
    out = jax.ShapeDtypeStruct((n, HALF), jnp.float32)
    return pl.pallas_call(
        _step_body,
        grid=(grid,),
        in_specs=[hblk, hblk, hblk, hblk, wblk, wblk, wblk, wblk,
                  vblk, vblk, vblk],
        out_specs=[hblk, hblk],
        out_shape=[out, out],
    )(h0, h1, a0, a1, W1, W2, W3, W4, b, g, be)


def _final_body(h0b, h1b, W1, W2, b, g, be, out, *, n_nodes):
    y = jnp.dot(h0b[...], W1[...], preferred_element_type=jnp.float32)
    y += jnp.dot(h1b[...], W2[...], preferred_element_type=jnp.float32)
    y = _gelu(_ln(y + b[...], g[...], be[...]))
    psum = jnp.sum(y, axis=0, keepdims=True) * (1.0 / n_nodes)
    pmax = jnp.max(y, axis=0, keepdims=True)
    part = jnp.concatenate([psum, pmax], axis=-1)
    i = pl.program_id(0)

    @pl.when(i == 0)
    def _():
        out[...] = part

    @pl.when(i > 0)
    def _():
        prev = out[...]
        out[...] = jnp.concatenate(
            [prev[:, :MOT] + part[:, :MOT],
             jnp.maximum(prev[:, MOT:], part[:, MOT:])], axis=-1)


def _final(h0, h1, W1, W2, b, g, be):
    n = h0.shape[0]
    grid = n // ROWBLK
    hblk = pl.BlockSpec((ROWBLK, HALF), lambda i: (i, 0))
    wblk = pl.BlockSpec((HALF, MOT), lambda i: (0, 0))
    vblk = pl.BlockSpec((1, MOT), lambda i: (0, 0))
    return pl.pallas_call(
        functools.partial(_final_body, n_nodes=n),
        grid=(grid,),
        in_specs=[hblk, hblk, wblk, wblk, vblk, vblk, vblk],
        out_specs=pl.BlockSpec((1, 2 * MOT), lambda i: (0, 0)),
        out_shape=jax.ShapeDtypeStruct((1, 2 * MOT), jnp.float32),
    )(h0, h1, W1, W2, b, g, be)


def kernel(x, edge_index, Wp, bp, gp, bep, Wa, ba, ga, bea, Wo, bo, go, beo):
    n, _ = x.shape
    n_edges = edge_index.shape[1]
    steps = Wa.shape[0]
    n_pad = ((n + 8 * NS - 1) // (8 * NS)) * (8 * NS)

    row = edge_index[0].astype(jnp.int32)
    col = edge_index[1].astype(jnp.int32)
    cap = NS * NCHUNK * CHUNK
    pad = cap - n_edges
    if pad:
        col = jnp.concatenate([col, jnp.zeros((pad,), jnp.int32)])
        trash = n + (jnp.arange(pad, dtype=jnp.int32) % (n_pad - n))
        row = jnp.concatenate([row, trash])
    colr = col.reshape(NS, NCHUNK, CHUNK)
    rowr = row.reshape(NS, NCHUNK, CHUNK)
    zeros = jnp.zeros((n_pad // NS, HALF), jnp.float32)

    r2 = lambda v: v.reshape(1, -1)
    h0, h1 = _proj(x, Wp, r2(bp), r2(gp), r2(bep))
    agg = _make_agg(n_pad)
    for s in range(steps):
        a0, a1 = agg(h0, h1, colr, rowr, zeros)
        Ws = Wa[s]
        h0, h1 = _step(h0, h1, a0, a1,
                       Ws[0 * HALF:1 * HALF], Ws[1 * HALF:2 * HALF],
                       Ws[2 * HALF:3 * HALF], Ws[3 * HALF:4 * HALF],
                       r2(ba[s]), r2(ga[s]), r2(bea[s]))
    return _final(h0, h1, Wo[:HALF], Wo[HALF:],
                  r2(bo), r2(go), r2(beo))


# speedup vs baseline: 3.6740x; 1.1597x over previous
import functools

import jax
import jax.numpy as jnp
from jax import lax
from jax.experimental import pallas as pl
from jax.experimental.pallas import tpu as pltpu
from jax.experimental.pallas import tpu_sc as plsc

MOT = 256
HALF = 128
NC = 2
NS = 16
CHUNK = 64
IW = 16
NCHUNK = 320
NBUF = 4
ROWBLK = 1000


def _ln(y, g, b, eps=1e-5):
    m = jnp.mean(y, axis=-1, keepdims=True)
    v = jnp.mean((y - m) ** 2, axis=-1, keepdims=True)
    return (y - m) * jax.lax.rsqrt(v + eps) * g + b


def _gelu(y):
    return 0.5 * y * (1.0 + lax.erf(y * (2.0 ** -0.5)))


def _make_agg(n_pad):
    rows_per_tile = n_pad // NS
    nw = NCHUNK // IW

    mesh = plsc.VectorSubcoreMesh(core_axis_name="c", subcore_axis_name="s")

    def body(h0, h1, colr, rowr, zeros, a0, a1,
             colw, roww, gbuf, aggsp, isem, gsem, ssem):
        c = lax.axis_index("c")
        s = lax.axis_index("s")
        sl = pl.ds(s * rows_per_tile, rows_per_tile)
        pltpu.sync_copy(zeros, aggsp.at[sl])
        plsc.subcore_barrier()

        def idx_dma(w):
            wb = lax.rem(w, 3)
            return (
                pltpu.make_async_copy(colr.at[s, pl.ds(w * IW, IW)], colw.at[wb], isem),
                pltpu.make_async_copy(rowr.at[s, pl.ds(w * IW, IW)], roww.at[wb], isem),
            )

        def run(h):
            def gather(k):
                wb, i = lax.rem(lax.div(k, IW), 3), lax.rem(k, IW)
                return pltpu.make_async_copy(
                    h.at[colw.at[wb, i]], gbuf.at[lax.rem(k, NBUF)], gsem)

            def scatter(k):
                wb, i = lax.rem(lax.div(k, IW), 3), lax.rem(k, IW)
                return pltpu.make_async_copy(
                    gbuf.at[lax.rem(k, NBUF)], aggsp.at[roww.at[wb, i]], ssem)

            for d in idx_dma(0):
                d.start()

            def step(j, carry):
                @pl.when(lax.rem(j, IW) == 0)
                def _():
                    w = lax.div(j, IW)

                    @pl.when(w < nw)
                    def _():
                        for d in idx_dma(w):
                            d.wait()

                    @pl.when(w + 1 < nw)
                    def _():
                        for d in idx_dma(w + 1):
                            d.start()

                @pl.when((j >= NBUF) & (j - NBUF < NCHUNK))
                def _():
                    scatter(j - NBUF).wait()

                @pl.when(j < NCHUNK)
                def _():
                    gather(j).start()

                @pl.when((j >= 2) & (j - 2 < NCHUNK))
                def _():
                    gather(j - 2).wait()
                    scatter(j - 2).start(add=True)
                return carry

            lax.fori_loop(0, NCHUNK + NBUF, step, 0, unroll=False)

        @pl.when(c == 0)
        def _():
            run(h0)

        @pl.when(c == 1)
        def _():
            run(h1)

        plsc.subcore_barrier()

        @pl.when(c == 0)
        def _():
            pltpu.sync_copy(aggsp.at[sl], a0.at[sl])

        @pl.when(c == 1)
        def _():
            pltpu.sync_copy(aggsp.at[sl], a1.at[sl])

    out = jax.ShapeDtypeStruct((n_pad, HALF), jnp.float32)
    return pl.kernel(
        body,
        out_type=[out, out],
        mesh=mesh,
        scratch_types=[
            pltpu.VMEM((3, IW, CHUNK), jnp.int32),
            pltpu.VMEM((3, IW, CHUNK), jnp.int32),
            pltpu.VMEM((NBUF, CHUNK, HALF), jnp.float32),
            pltpu.VMEM_SHARED((n_pad, HALF), jnp.float32),
            pltpu.SemaphoreType.DMA,
            pltpu.SemaphoreType.DMA,
            pltpu.SemaphoreType.DMA,
        ],
    )


def _proj_body(xb, Wp, bp, gp, bep, h0, h1):
    y = jnp.dot(xb[...], Wp[...], preferred_element_type=jnp.float32)
    y = _gelu(_ln(y + bp[...], gp[...], bep[...]))
    h0[...] = y[:, :HALF]
    h1[...] = y[:, HALF:]


def _proj(x, Wp, bp, gp, bep):
    n, hid = x.shape
    grid = n // ROWBLK
    out = jax.ShapeDtypeStruct((n, HALF), jnp.float32)
    hblk = pl.BlockSpec((ROWBLK, HALF), lambda i: (i, 0))
    vblk = pl.BlockSpec((1, MOT), lambda i: (0, 0))
    return pl.pallas_call(
        _proj_body,
        grid=(grid,),
        in_specs=[
            pl.BlockSpec((ROWBLK, hid), lambda i: (i, 0)),
            pl.BlockSpec((hid, MOT), lambda i: (0, 0)),
            vblk, vblk, vblk,
        ],
        out_specs=[hblk, hblk],
        out_shape=[out, out],
    )(x, Wp, bp, gp, bep)


def _step_body(h0b, h1b, a0b, a1b, W1, W2, W3, W4, b, g, be, o0, o1):
    y = jnp.dot(h0b[...], W1[...], preferred_element_type=jnp.float32)
    y += jnp.dot(h1b[...], W2[...], preferred_element_type=jnp.float32)
    y += jnp.dot(a0b[...], W3[...], preferred_element_type=jnp.float32)
    y += jnp.dot(a1b[...], W4[...], preferred_element_type=jnp.float32)
    y = _gelu(_ln(y + b[...], g[...], be[...]))
    o0[...] = y[:, :HALF]
    o1[...] = y[:, HALF:]


def _step(h0, h1, a0, a1, W1, W2, W3, W4, b, g, be):
    n = h0.shape[0]
    grid = n // ROWBLK
    hblk = pl.BlockSpec((ROWBLK, HALF), lambda i: (i, 0))
    wblk = pl.BlockSpec((HALF, MOT), lambda i: (0, 0))
    vblk = pl.BlockSpec((1, MOT), lambda i: (0, 0))
    out = jax.ShapeDtypeStruct((n, HALF), jnp.float32)
    return pl.pallas_call(
        _step_body,
        grid=(grid,),
        in_specs=[hblk, hblk, hblk, hblk, wblk, wblk, wblk, wblk,
                  vblk, vblk, vblk],
        out_specs=[hblk, hblk],
        out_shape=[out, out],
    )(h0, h1, a0, a1, W1, W2, W3, W4, b, g, be)


def _final_body(h0b, h1b, W1, W2, b, g, be, out, *, n_nodes):
    y = jnp.dot(h0b[...], W1[...], preferred_element_type=jnp.float32)
    y += jnp.dot(h1b[...], W2[...], preferred_element_type=jnp.float32)
    y = _gelu(_ln(y + b[...], g[...], be[...]))
    psum = jnp.sum(y, axis=0, keepdims=True) * (1.0 / n_nodes)
    pmax = jnp.max(y, axis=0, keepdims=True)
    part = jnp.concatenate([psum, pmax], axis=-1)
    i = pl.program_id(0)

    @pl.when(i == 0)
    def _():
        out[...] = part

    @pl.when(i > 0)
    def _():
        prev = out[...]
        out[...] = jnp.concatenate(
            [prev[:, :MOT] + part[:, :MOT],
             jnp.maximum(prev[:, MOT:], part[:, MOT:])], axis=-1)


def _final(h0, h1, W1, W2, b, g, be):
    n = h0.shape[0]
    grid = n // ROWBLK
    hblk = pl.BlockSpec((ROWBLK, HALF), lambda i: (i, 0))
    wblk = pl.BlockSpec((HALF, MOT), lambda i: (0, 0))
    vblk = pl.BlockSpec((1, MOT), lambda i: (0, 0))
    return pl.pallas_call(
        functools.partial(_final_body, n_nodes=n),
        grid=(grid,),
        in_specs=[hblk, hblk, wblk, wblk, vblk, vblk, vblk],
        out_specs=pl.BlockSpec((1, 2 * MOT), lambda i: (0, 0)),
        out_shape=jax.ShapeDtypeStruct((1, 2 * MOT), jnp.float32),
    )(h0, h1, W1, W2, b, g, be)


def kernel(x, edge_index, Wp, bp, gp, bep, Wa, ba, ga, bea, Wo, bo, go, beo):
    n, _ = x.shape
    n_edges = edge_index.shape[1]
    steps = Wa.shape[0]
    n_pad = ((n + 8 * NS - 1) // (8 * NS)) * (8 * NS)

    row = edge_index[0].astype(jnp.int32)
    col = edge_index[1].astype(jnp.int32)
    cap = NS * NCHUNK * CHUNK
    pad = cap - n_edges
    if pad:
        col = jnp.concatenate([col, jnp.zeros((pad,), jnp.int32)])
        trash = n + (jnp.arange(pad, dtype=jnp.int32) % (n_pad - n))
        row = jnp.concatenate([row, trash])
    colr = col.reshape(NS, NCHUNK, CHUNK)
    rowr = row.reshape(NS, NCHUNK, CHUNK)
    zeros = jnp.zeros((n_pad // NS, HALF), jnp.float32)

    r2 = lambda v: v.reshape(1, -1)
    h0, h1 = _proj(x, Wp, r2(bp), r2(gp), r2(bep))
    agg = _make_agg(n_pad)
    for s in range(steps):
        a0, a1 = agg(h0, h1, colr, rowr, zeros)
        Ws = Wa[s]
        h0, h1 = _step(h0, h1, a0, a1,
                       Ws[0 * HALF:1 * HALF], Ws[1 * HALF:2 * HALF],
                       Ws[2 * HALF:3 * HALF], Ws[3 * HALF:4 * HALF],
                       r2(ba[s]), r2(ga[s]), r2(bea[s]))
    return _final(h0, h1, Wo[:HALF], Wo[HALF:],
                  r2(bo), r2(go), r2(beo))
